# baseline (speedup 1.0000x reference)
import functools

import jax
import jax.numpy as jnp
from jax import lax
from jax.experimental import pallas as pl
from jax.experimental.pallas import tpu as pltpu
from jax.experimental.pallas import tpu_sc as plsc

H = 56
W = 56
C = 256
NROI = 512
P = 7
NBIN = P * P
NSPAN = 9
NTAB = NSPAN * 4
HW = H * W

NC = 2
NS = 16
NWORK = NC * NS
RPW = NROI // NWORK
GROWS = 104
NBUF = 4


def _tables_body(fm_ref, out_ref):
    s = pl.program_id(0)
    ii = lax.broadcasted_iota(jnp.int32, (C, C), 0)
    jj = lax.broadcasted_iota(jnp.int32, (C, C), 1)
    eye = (ii == jj).astype(jnp.float32)
    F = lax.dot_general(fm_ref[...], eye, (((0,), (0,)), ((), ())),
                        precision=lax.Precision.HIGHEST)
    X = F
    for j in range(1, NSPAN):
        sh = jnp.concatenate(
            [F[j * W:], jnp.broadcast_to(F[-1:], (j * W, C))], axis=0)
        X = jnp.where(s >= j, jnp.maximum(X, sh), X)
    Y = X
    for kw in range(4):
        if kw > 0:
            d = 1 << (kw - 1)
            sh = jnp.concatenate(
                [Y[d:], jnp.broadcast_to(Y[-1:], (d, C))], axis=0)
            Y = jnp.maximum(Y, sh)
        out_ref[0, kw] = Y


def _build_tables(FM2):
    return pl.pallas_call(
        _tables_body,
        grid=(NSPAN,),
        in_specs=[pl.BlockSpec((C, HW), lambda g: (0, 0))],
        out_specs=pl.BlockSpec((1, 4, HW, C), lambda g: (g, 0, 0, 0)),
        out_shape=jax.ShapeDtypeStruct((NSPAN, 4, HW, C), jnp.float32),
    )(FM2)


def _rint_nonneg(x):
    fl = x.astype(jnp.int32)
    fr = x - fl.astype(jnp.float32)
    odd = (fl & 1) == 1
    up = (fr > 0.5) | ((fr == 0.5) & odd)
    return fl + up.astype(jnp.int32)


def _sc_body(tabs, rois, out, rv, idxb, rows, outT,
             g0, g1, g2, g3, osem):
    cid = lax.axis_index("c")
    sid = lax.axis_index("s")
    wid = sid * NC + cid
    base = wid * RPW

    pltpu.sync_copy(rois.at[pl.ds(base * 1, RPW)], rv)

    lane = jnp.arange(RPW, dtype=jnp.int32)
    zero = jnp.zeros((RPW,), jnp.int32)

    def rcol(d):
        return plsc.load_gather(rv, [lane, jnp.full((RPW,), d, jnp.int32)])

    def idx_store(slot, val):
        plsc.store_scatter(
            idxb,
            [lane, zero, jnp.full((RPW,), slot, jnp.int32)],
            val)

    for k in range(2 * NBIN, GROWS):
        idx_store(k, zero)

    y0 = jnp.clip(_rint_nonneg(rcol(0) * float(H)), 0, H - 1)
    x0 = jnp.clip(_rint_nonneg(rcol(1) * float(W)), 0, W - 1)
    rh = jnp.minimum(jnp.maximum(_rint_nonneg(rcol(2) * float(H)), 1), H - y0)
    rw = jnp.minimum(jnp.maximum(_rint_nonneg(rcol(3) * float(W)), 1), W - x0)

    def edges(p, v0, rv_):
        st = v0 + (p * rv_) // P
        e = v0 + ((p + 1) * rv_ + (P - 1)) // P
        e = jnp.maximum(e, st + 1)
        return st, e - st

    hA = [edges(p, y0, rh) for p in range(P)]
    wA = []
    for p in range(P):
        ws, dw = edges(p, x0, rw)
        pw = jnp.where(dw >= 8, 8,
                       jnp.where(dw >= 4, 4, jnp.where(dw >= 2, 2, 1)))
        kw = ((dw >= 2).astype(jnp.int32) + (dw >= 4).astype(jnp.int32)
              + (dw >= 8).astype(jnp.int32))
        wA.append((ws, kw, (ws + dw) - pw))

    for py in range(P):
        hs, dh = hA[py]
        rbase = (dh - 1) * (4 * HW) + hs * W
        for px in range(P):
            ws, kw, c1 = wA[px]
            tb = rbase + kw * HW
            b = py * P + px
            idx_store(b, tb + ws)
            idx_store(NBIN + b, tb + c1)

    gsems = (g0, g1, g2, g3)
    ivec = jnp.arange(16, dtype=jnp.int32) * NBIN

    def issue(r, buf):
        pltpu.async_copy(tabs.at[idxb.at[r, 0]], rows.at[buf], gsems[buf])

    def drain(buf):
        pltpu.make_async_copy(
            tabs.at[pl.ds(0, GROWS)], rows.at[buf], gsems[buf]).wait()

    def owait():
        pltpu.make_async_copy(outT, out.at[0], osem).wait()

    def compute(r, buf):
        def per_row(py, carry):
            b0 = py * P
            for px in range(P):
                bvec = ivec + (b0 + px)
                for v in range(C // 16):
                    m = jnp.maximum(
                        rows[buf, b0 + px, pl.ds(16 * v, 16)],
                        rows[buf, NBIN + b0 + px, pl.ds(16 * v, 16)])
                    plsc.store_scatter(outT, [bvec + (16 * NBIN * v)], m)
            return carry

        lax.fori_loop(0, P, per_row, 0)
        pltpu.async_copy(outT, out.at[base + r], osem)

    issue(0, 0)
    issue(1, 1)

    def quad(g, carry):
        r0 = 4 * g
        for i in range(NBUF):
            r = r0 + i
            drain(i)

            @pl.when(r + 2 < RPW)
            def _():
                issue(r + 2, (i + 2) % NBUF)

            @pl.when(r >= 1)
            def _():
                owait()

            compute(r, i)
        return carry

    lax.fori_loop(0, RPW // NBUF, quad, 0)
    owait()


def _sc_pool(tabs, rois):
    mesh = plsc.VectorSubcoreMesh(core_axis_name="c", subcore_axis_name="s")
    return pl.kernel(
        _sc_body,
        mesh=mesh,
        compiler_params=pltpu.CompilerParams(needs_layout_passes=False),
        out_type=jax.ShapeDtypeStruct((NROI, C * NBIN), jnp.float32),
        scratch_types=[
            pltpu.VMEM((RPW, 4), jnp.float32),
            pltpu.VMEM((RPW, 1, GROWS), jnp.int32),
            pltpu.VMEM((NBUF, GROWS, C), jnp.float32),
            pltpu.VMEM((C * NBIN,), jnp.float32),
            pltpu.SemaphoreType.DMA,
            pltpu.SemaphoreType.DMA,
            pltpu.SemaphoreType.DMA,
            pltpu.SemaphoreType.DMA,
            pltpu.SemaphoreType.DMA,
        ],
    )(tabs, rois)


@jax.jit
def kernel(FM, rois):
    tabs = _build_tables(FM.reshape(C, HW)).reshape(NTAB * HW, C)
    out = _sc_pool(tabs, rois)
    return out.reshape(NROI, C, P, P)


# speedup vs baseline: 1.0453x; 1.0453x over previous
import functools

import jax
import jax.numpy as jnp
from jax import lax
from jax.experimental import pallas as pl
from jax.experimental.pallas import tpu as pltpu
from jax.experimental.pallas import tpu_sc as plsc

H = 56
W = 56
C = 256
NROI = 512
P = 7
NBIN = P * P
NSPAN = 9
NTAB = NSPAN * 4
HW = H * W

NC = 2
NS = 16
NWORK = NC * NS
RPW = NROI // NWORK
GROWS = 104
NBUF = 4


def _transpose_body(fm_ref, out_ref):
    ii = lax.broadcasted_iota(jnp.int32, (C, C), 0)
    jj = lax.broadcasted_iota(jnp.int32, (C, C), 1)
    eye = (ii == jj).astype(jnp.float32)
    out_ref[...] = lax.dot_general(
        fm_ref[...], eye, (((0,), (0,)), ((), ())),
        precision=lax.Precision.HIGHEST)


def _transpose(FM2):
    return pl.pallas_call(
        _transpose_body,
        out_shape=jax.ShapeDtypeStruct((HW, C), jnp.float32),
    )(FM2)


def _tables_body(fmt_ref, out_ref):
    s = pl.program_id(0)
    F = fmt_ref[...]
    X = F
    for j in range(1, NSPAN):
        sh = jnp.concatenate(
            [F[j * W:], jnp.broadcast_to(F[-1:], (j * W, C))], axis=0)
        X = jnp.where(s >= j, jnp.maximum(X, sh), X)
    Y = X
    for kw in range(4):
        if kw > 0:
            d = 1 << (kw - 1)
            sh = jnp.concatenate(
                [Y[d:], jnp.broadcast_to(Y[-1:], (d, C))], axis=0)
            Y = jnp.maximum(Y, sh)
        out_ref[0, kw] = Y


def _build_tables(FMt):
    return pl.pallas_call(
        _tables_body,
        grid=(NSPAN,),
        in_specs=[pl.BlockSpec((HW, C), lambda g: (0, 0))],
        out_specs=pl.BlockSpec((1, 4, HW, C), lambda g: (g, 0, 0, 0)),
        out_shape=jax.ShapeDtypeStruct((NSPAN, 4, HW, C), jnp.float32),
    )(FMt)


def _rint_nonneg(x):
    fl = x.astype(jnp.int32)
    fr = x - fl.astype(jnp.float32)
    odd = (fl & 1) == 1
    up = (fr > 0.5) | ((fr == 0.5) & odd)
    return fl + up.astype(jnp.int32)


def _sc_body(tabs, rois, out, rv, idxb, rows, outT,
             g0, g1, g2, g3, osem):
    cid = lax.axis_index("c")
    sid = lax.axis_index("s")
    wid = sid * NC + cid
    base = wid * RPW

    pltpu.sync_copy(rois.at[pl.ds(base * 1, RPW)], rv)

    lane = jnp.arange(RPW, dtype=jnp.int32)
    zero = jnp.zeros((RPW,), jnp.int32)

    def rcol(d):
        return plsc.load_gather(rv, [lane, jnp.full((RPW,), d, jnp.int32)])

    def idx_store(slot, val):
        plsc.store_scatter(
            idxb,
            [lane, zero, jnp.full((RPW,), slot, jnp.int32)],
            val)

    for k in range(2 * NBIN, GROWS):
        idx_store(k, zero)

    y0 = jnp.clip(_rint_nonneg(rcol(0) * float(H)), 0, H - 1)
    x0 = jnp.clip(_rint_nonneg(rcol(1) * float(W)), 0, W - 1)
    rh = jnp.minimum(jnp.maximum(_rint_nonneg(rcol(2) * float(H)), 1), H - y0)
    rw = jnp.minimum(jnp.maximum(_rint_nonneg(rcol(3) * float(W)), 1), W - x0)

    def edges(p, v0, rv_):
        st = v0 + (p * rv_) // P
        e = v0 + ((p + 1) * rv_ + (P - 1)) // P
        e = jnp.maximum(e, st + 1)
        return st, e - st

    hA = [edges(p, y0, rh) for p in range(P)]
    wA = []
    for p in range(P):
        ws, dw = edges(p, x0, rw)
        pw = jnp.where(dw >= 8, 8,
                       jnp.where(dw >= 4, 4, jnp.where(dw >= 2, 2, 1)))
        kw = ((dw >= 2).astype(jnp.int32) + (dw >= 4).astype(jnp.int32)
              + (dw >= 8).astype(jnp.int32))
        wA.append((ws, kw, (ws + dw) - pw))

    for py in range(P):
        hs, dh = hA[py]
        rbase = (dh - 1) * (4 * HW) + hs * W
        for px in range(P):
            ws, kw, c1 = wA[px]
            tb = rbase + kw * HW
            b = py * P + px
            idx_store(b, tb + ws)
            idx_store(NBIN + b, tb + c1)

    gsems = (g0, g1, g2, g3)
    ivec = jnp.arange(16, dtype=jnp.int32) * NBIN

    def issue(r, buf):
        pltpu.async_copy(tabs.at[idxb.at[r, 0]], rows.at[buf], gsems[buf])

    def drain(buf):
        pltpu.make_async_copy(
            tabs.at[pl.ds(0, GROWS)], rows.at[buf], gsems[buf]).wait()

    def owait():
        pltpu.make_async_copy(outT, out.at[0], osem).wait()

    def compute(r, buf):
        def per_row(py, carry):
            b0 = py * P
            for px in range(P):
                bvec = ivec + (b0 + px)
                for v in range(C // 16):
                    m = jnp.maximum(
                        rows[buf, b0 + px, pl.ds(16 * v, 16)],
                        rows[buf, NBIN + b0 + px, pl.ds(16 * v, 16)])
                    plsc.store_scatter(outT, [bvec + (16 * NBIN * v)], m)
            return carry

        lax.fori_loop(0, P, per_row, 0)
        pltpu.async_copy(outT, out.at[base + r], osem)

    issue(0, 0)
    issue(1, 1)

    def quad(g, carry):
        r0 = 4 * g
        for i in range(NBUF):
            r = r0 + i
            drain(i)

            @pl.when(r + 2 < RPW)
            def _():
                issue(r + 2, (i + 2) % NBUF)

            @pl.when(r >= 1)
            def _():
                owait()

            compute(r, i)
        return carry

    lax.fori_loop(0, RPW // NBUF, quad, 0)
    owait()


def _sc_pool(tabs, rois):
    mesh = plsc.VectorSubcoreMesh(core_axis_name="c", subcore_axis_name="s")
    return pl.kernel(
        _sc_body,
        mesh=mesh,
        compiler_params=pltpu.CompilerParams(needs_layout_passes=False),
        out_type=jax.ShapeDtypeStruct((NROI, C * NBIN), jnp.float32),
        scratch_types=[
            pltpu.VMEM((RPW, 4), jnp.float32),
            pltpu.VMEM((RPW, 1, GROWS), jnp.int32),
            pltpu.VMEM((NBUF, GROWS, C), jnp.float32),
            pltpu.VMEM((C * NBIN,), jnp.float32),
            pltpu.SemaphoreType.DMA,
            pltpu.SemaphoreType.DMA,
            pltpu.SemaphoreType.DMA,
            pltpu.SemaphoreType.DMA,
            pltpu.SemaphoreType.DMA,
        ],
    )(tabs, rois)


@jax.jit
def kernel(FM, rois):
    tabs = _build_tables(_transpose(FM.reshape(C, HW))).reshape(NTAB * HW, C)
    out = _sc_pool(tabs, rois)
    return out.reshape(NROI, C, P, P)


# speedup vs baseline: 1.0819x; 1.0350x over previous
import functools

import jax
import jax.numpy as jnp
from jax import lax
from jax.experimental import pallas as pl
from jax.experimental.pallas import tpu as pltpu
from jax.experimental.pallas import tpu_sc as plsc

H = 56
W = 56
C = 256
NROI = 512
P = 7
NBIN = P * P
NSPAN = 9
NTAB = NSPAN * 4
HW = H * W

NC = 2
NS = 16
NWORK = NC * NS
RPW = NROI // NWORK
GROWS = 104
NBUF = 4


def _tables_body(fmt_ref, out_ref):
    s = pl.program_id(0)
    F = fmt_ref[...]
    X = F
    for j in range(1, NSPAN):
        sh = jnp.concatenate(
            [F[j * W:], jnp.broadcast_to(F[-1:], (j * W, C))], axis=0)
        X = jnp.where(s >= j, jnp.maximum(X, sh), X)
    Y = X
    for kw in range(4):
        if kw > 0:
            d = 1 << (kw - 1)
            sh = jnp.concatenate(
                [Y[d:], jnp.broadcast_to(Y[-1:], (d, C))], axis=0)
            Y = jnp.maximum(Y, sh)
        out_ref[0, kw] = Y


def _build_tables(FMt):
    return pl.pallas_call(
        _tables_body,
        grid=(NSPAN,),
        in_specs=[pl.BlockSpec((HW, C), lambda g: (0, 0))],
        out_specs=pl.BlockSpec((1, 4, HW, C), lambda g: (g, 0, 0, 0)),
        out_shape=jax.ShapeDtypeStruct((NSPAN, 4, HW, C), jnp.float32),
    )(FMt)


def _rint_nonneg(x):
    fl = x.astype(jnp.int32)
    fr = x - fl.astype(jnp.float32)
    odd = (fl & 1) == 1
    up = (fr > 0.5) | ((fr == 0.5) & odd)
    return fl + up.astype(jnp.int32)


def _sc_body(tabs, rois, out, rv, idxb, rows, outT,
             g0, g1, g2, g3, osem):
    cid = lax.axis_index("c")
    sid = lax.axis_index("s")
    wid = sid * NC + cid
    base = wid * RPW

    pltpu.sync_copy(rois.at[pl.ds(base * 1, RPW)], rv)

    lane = jnp.arange(RPW, dtype=jnp.int32)
    zero = jnp.zeros((RPW,), jnp.int32)

    def rcol(d):
        return plsc.load_gather(rv, [lane, jnp.full((RPW,), d, jnp.int32)])

    def idx_store(slot, val):
        plsc.store_scatter(
            idxb,
            [lane, zero, jnp.full((RPW,), slot, jnp.int32)],
            val)

    for k in range(2 * NBIN, GROWS):
        idx_store(k, zero)

    y0 = jnp.clip(_rint_nonneg(rcol(0) * float(H)), 0, H - 1)
    x0 = jnp.clip(_rint_nonneg(rcol(1) * float(W)), 0, W - 1)
    rh = jnp.minimum(jnp.maximum(_rint_nonneg(rcol(2) * float(H)), 1), H - y0)
    rw = jnp.minimum(jnp.maximum(_rint_nonneg(rcol(3) * float(W)), 1), W - x0)

    def edges(p, v0, rv_):
        st = v0 + (p * rv_) // P
        e = v0 + ((p + 1) * rv_ + (P - 1)) // P
        e = jnp.maximum(e, st + 1)
        return st, e - st

    hA = [edges(p, y0, rh) for p in range(P)]
    wA = []
    for p in range(P):
        ws, dw = edges(p, x0, rw)
        pw = jnp.where(dw >= 8, 8,
                       jnp.where(dw >= 4, 4, jnp.where(dw >= 2, 2, 1)))
        kw = ((dw >= 2).astype(jnp.int32) + (dw >= 4).astype(jnp.int32)
              + (dw >= 8).astype(jnp.int32))
        wA.append((ws, kw, (ws + dw) - pw))

    for py in range(P):
        hs, dh = hA[py]
        rbase = (dh - 1) * (4 * HW) + hs * W
        for px in range(P):
            ws, kw, c1 = wA[px]
            tb = rbase + kw * HW
            b = py * P + px
            idx_store(b, tb + ws)
            idx_store(NBIN + b, tb + c1)

    gsems = (g0, g1, g2, g3)
    ivec = jnp.arange(16, dtype=jnp.int32) * NBIN

    def issue(r, buf):
        pltpu.async_copy(tabs.at[idxb.at[r, 0]], rows.at[buf], gsems[buf])

    def drain(buf):
        pltpu.make_async_copy(
            tabs.at[pl.ds(0, GROWS)], rows.at[buf], gsems[buf]).wait()

    def owait():
        pltpu.make_async_copy(outT, out.at[0], osem).wait()

    def compute(r, buf):
        def per_row(py, carry):
            b0 = py * P
            for px in range(P):
                bvec = ivec + (b0 + px)
                for v in range(C // 16):
                    m = jnp.maximum(
                        rows[buf, b0 + px, pl.ds(16 * v, 16)],
                        rows[buf, NBIN + b0 + px, pl.ds(16 * v, 16)])
                    plsc.store_scatter(outT, [bvec + (16 * NBIN * v)], m)
            return carry

        lax.fori_loop(0, P, per_row, 0)
        pltpu.async_copy(outT, out.at[base + r], osem)

    issue(0, 0)
    issue(1, 1)

    def quad(g, carry):
        r0 = 4 * g
        for i in range(NBUF):
            r = r0 + i
            drain(i)

            @pl.when(r + 2 < RPW)
            def _():
                issue(r + 2, (i + 2) % NBUF)

            @pl.when(r >= 1)
            def _():
                owait()

            compute(r, i)
        return carry

    lax.fori_loop(0, RPW // NBUF, quad, 0)
    owait()


def _sc_pool(tabs, rois):
    mesh = plsc.VectorSubcoreMesh(core_axis_name="c", subcore_axis_name="s")
    return pl.kernel(
        _sc_body,
        mesh=mesh,
        compiler_params=pltpu.CompilerParams(needs_layout_passes=False),
        out_type=jax.ShapeDtypeStruct((NROI, C * NBIN), jnp.float32),
        scratch_types=[
            pltpu.VMEM((RPW, 4), jnp.float32),
            pltpu.VMEM((RPW, 1, GROWS), jnp.int32),
            pltpu.VMEM((NBUF, GROWS, C), jnp.float32),
            pltpu.VMEM((C * NBIN,), jnp.float32),
            pltpu.SemaphoreType.DMA,
            pltpu.SemaphoreType.DMA,
            pltpu.SemaphoreType.DMA,
            pltpu.SemaphoreType.DMA,
            pltpu.SemaphoreType.DMA,
        ],
    )(tabs, rois)


@jax.jit
def kernel(FM, rois):
    FMt = jnp.transpose(FM, (1, 2, 0)).reshape(HW, C)
    tabs = _build_tables(FMt).reshape(NTAB * HW, C)
    out = _sc_pool(tabs, rois)
    return out.reshape(NROI, C, P, P)


# speedup vs baseline: 1.0909x; 1.0083x over previous
import functools

import jax
import jax.numpy as jnp
from jax import lax
from jax.experimental import pallas as pl
from jax.experimental.pallas import tpu as pltpu
from jax.experimental.pallas import tpu_sc as plsc

H = 56
W = 56
C = 256
NROI = 512
P = 7
NBIN = P * P
NSPAN = 9
NTAB = NSPAN * 4
HW = H * W

NC = 2
NS = 16
NWORK = NC * NS
RPW = NROI // NWORK
GROWS = 104


def _tables_body(fmt_ref, out_ref):
    s = pl.program_id(0)
    F = fmt_ref[...]
    X = F
    for j in range(1, NSPAN):
        sh = jnp.concatenate(
            [F[j:], jnp.broadcast_to(F[-1:], (j, W, C))], axis=0)
        X = jnp.where(s >= j, jnp.maximum(X, sh), X)
    Y = X
    for kw in range(4):
        if kw > 0:
            d = 1 << (kw - 1)
            sh = jnp.concatenate(
                [Y[:, d:], jnp.broadcast_to(Y[:, -1:], (H, d, C))], axis=1)
            Y = jnp.maximum(Y, sh)
        out_ref[0, kw] = Y


def _build_tables(FMt):
    return pl.pallas_call(
        _tables_body,
        grid=(NSPAN,),
        in_specs=[pl.BlockSpec((H, W, C), lambda g: (0, 0, 0))],
        out_specs=pl.BlockSpec((1, 4, H, W, C), lambda g: (g, 0, 0, 0, 0)),
        out_shape=jax.ShapeDtypeStruct((NSPAN, 4, H, W, C), jnp.float32),
    )(FMt)


def _rint_nonneg(x):
    fl = x.astype(jnp.int32)
    fr = x - fl.astype(jnp.float32)
    odd = (fl & 1) == 1
    up = (fr > 0.5) | ((fr == 0.5) & odd)
    return fl + up.astype(jnp.int32)


def _sc_body(tabs, roist, out, rv, idxb, rows, outT, semA, semB):
    cid = lax.axis_index("c")
    sid = lax.axis_index("s")
    wid = sid * NC + cid
    base = wid * RPW

    for d in range(4):
        pltpu.sync_copy(roist.at[d, pl.ds(base * 1, RPW)], rv.at[d])

    lane = jnp.arange(RPW, dtype=jnp.int32)
    zero = jnp.zeros((RPW,), jnp.int32)

    def idx_store(slot, val):
        plsc.store_scatter(
            idxb,
            [lane, zero, jnp.full((RPW,), slot, jnp.int32)],
            val)

    for k in range(2 * NBIN, GROWS):
        idx_store(k, zero)

    fi = rv[0]
    fj = rv[1]
    fh = rv[2]
    fw = rv[3]
    y0 = jnp.clip(_rint_nonneg(fi * float(H)), 0, H - 1)
    x0 = jnp.clip(_rint_nonneg(fj * float(W)), 0, W - 1)
    rh = jnp.minimum(jnp.maximum(_rint_nonneg(fh * float(H)), 1), H - y0)
    rw = jnp.minimum(jnp.maximum(_rint_nonneg(fw * float(W)), 1), W - x0)

    def edges(p, v0, rv_):
        st = v0 + (p * rv_) // P
        e = v0 + ((p + 1) * rv_ + (P - 1)) // P
        e = jnp.maximum(e, st + 1)
        return st, e - st

    hA = [edges(p, y0, rh) for p in range(P)]
    wA = []
    for p in range(P):
        ws, dw = edges(p, x0, rw)
        pw = jnp.where(dw >= 8, 8,
                       jnp.where(dw >= 4, 4, jnp.where(dw >= 2, 2, 1)))
        kw = ((dw >= 2).astype(jnp.int32) + (dw >= 4).astype(jnp.int32)
              + (dw >= 8).astype(jnp.int32))
        wA.append((ws, kw, (ws + dw) - pw))

    for py in range(P):
        hs, dh = hA[py]
        rbase = (dh - 1) * (4 * HW) + hs * W
        for px in range(P):
            ws, kw, c1 = wA[px]
            tb = rbase + kw * HW
            b = py * P + px
            idx_store(b, tb + ws)
            idx_store(NBIN + b, tb + c1)

    sems = (semA, semB)
    ivec = jnp.arange(16, dtype=jnp.int32) * NBIN

    def issue(r, buf):
        pltpu.async_copy(
            tabs.at[idxb.at[r, 0]],
            rows.at[buf], sems[buf])

    def drain(buf):
        pltpu.make_async_copy(
            tabs.at[pl.ds(0, GROWS)], rows.at[buf], sems[buf]).wait()

    def compute(r, buf):
        def per_row(py, carry):
            b0 = py * P
            for px in range(P):
                bvec = ivec + (b0 + px)
                for v in range(C // 16):
                    m = jnp.maximum(
                        rows[buf, b0 + px, pl.ds(16 * v, 16)],
                        rows[buf, NBIN + b0 + px, pl.ds(16 * v, 16)])
                    plsc.store_scatter(outT, [bvec + (16 * NBIN * v)], m)
            return carry

        lax.fori_loop(0, P, per_row, 0)
        pltpu.sync_copy(outT, out.at[base + r])

    issue(0, 0)
    issue(1, 1)

    def pair(g, carry):
        ra = 2 * g
        drain(0)
        compute(ra, 0)

        @pl.when(ra + 2 < RPW)
        def _():
            issue(ra + 2, 0)

        drain(1)
        compute(ra + 1, 1)

        @pl.when(ra + 3 < RPW)
        def _():
            issue(ra + 3, 1)

        return carry

    lax.fori_loop(0, RPW // 2, pair, 0)


def _sc_pool(tabs, roist):
    mesh = plsc.VectorSubcoreMesh(core_axis_name="c", subcore_axis_name="s")
    return pl.kernel(
        _sc_body,
        mesh=mesh,
        compiler_params=pltpu.CompilerParams(needs_layout_passes=False),
        out_type=jax.ShapeDtypeStruct((NROI, C * NBIN), jnp.float32),
        scratch_types=[
            pltpu.VMEM((4, RPW), jnp.float32),
            pltpu.VMEM((RPW, 1, GROWS), jnp.int32),
            pltpu.VMEM((2, GROWS, C), jnp.float32),
            pltpu.VMEM((C * NBIN,), jnp.float32),
            pltpu.SemaphoreType.DMA,
            pltpu.SemaphoreType.DMA,
        ],
    )(tabs, roist)


@jax.jit
def kernel(FM, rois):
    FMt = jnp.transpose(FM, (1, 2, 0))
    tabs = _build_tables(FMt).reshape(NTAB * HW, C)
    roist = jnp.transpose(rois, (1, 0))
    out = _sc_pool(tabs, roist)
    return out.reshape(NROI, C, P, P)
